# baseline (speedup 1.0000x reference)
import jax
import jax.numpy as jnp
from jax import lax
from jax.experimental import pallas as pl
from jax.experimental.pallas import tpu as pltpu
from jax.experimental.pallas import tpu_sc as plsc

_MAXLEN = 200
_EMBED = 64
_VOCAB = 100000
_BATCH = 4096
_LANES = 16
_PADDED = 128

_NC = 2
_NS = 16
_NW = _NC * _NS
_BPW = _BATCH // _NW
_ROUNDS = _BPW // 2
_ISTRIDE = 104
_J = _EMBED // _LANES


def _make_tec_body(tlen):
    n1 = min(tlen, 128)
    n2 = tlen - n1

    def _tec_body(x_hbm, tok_hbm, pos_hbm, out_hbm, pos_v, idx_v, rows_v,
                  outs_v, g0, g1, o0, o1, i0, i1):
        c = lax.axis_index("c")
        s = lax.axis_index("s")
        wid = s * _NC + c
        gsems = (g0, g1)
        osems = (o0, o1)
        isems = (i0, i1)
        pltpu.sync_copy(pos_hbm, pos_v)
        elem_base = wid * _BPW

        def stage_idx(sem, j, b):
            pltpu.async_copy(x_hbm.at[pl.ds(b * tlen, tlen)],
                             idx_v.at[pl.ds(pl.multiple_of(j * _ISTRIDE, 8),
                                            tlen)],
                             isems[sem])

        def fire(slot, j, b):
            j0 = pl.multiple_of(j * _ISTRIDE, 8)
            pltpu.make_async_copy(x_hbm.at[pl.ds(b * tlen, tlen)],
                                  idx_v.at[pl.ds(j0, tlen)],
                                  isems[slot]).wait()
            pltpu.async_copy(tok_hbm.at[idx_v.at[pl.ds(j0, n1)]],
                             rows_v.at[slot, pl.ds(0, n1)], gsems[slot])
            if n2:
                pltpu.async_copy(tok_hbm.at[idx_v.at[pl.ds(j0 + n1, n2)]],
                                 rows_v.at[slot, pl.ds(n1, n2)], gsems[slot])

        def wait_gathers(slot):
            pltpu.make_async_copy(tok_hbm.at[pl.ds(0, tlen)],
                                  rows_v.at[slot], gsems[slot]).wait()

        def drain_out(slot, b):
            pltpu.make_async_copy(outs_v.at[slot], out_hbm.at[b],
                                  osems[slot]).wait()

        def add_positions(slot):
            @plsc.parallel_loop(0, tlen, unroll=4)
            def _(t):
                for jj in range(_J):
                    p = pos_v[pl.ds(t * _EMBED + jj * _LANES, _LANES)]
                    outs_v[slot, t, pl.ds(jj * _LANES, _LANES)] = (
                        rows_v[slot, t, pl.ds(jj * _LANES, _LANES)] + p
                    )

        def proc(slot, b, drain_pred):
            wait_gathers(slot)

            @pl.when(drain_pred)
            def _():
                drain_out(slot, b - 2)

            add_positions(slot)
            pltpu.async_copy(outs_v.at[slot], out_hbm.at[b], osems[slot])

        def round_body(r, carry):
            b0 = elem_base + r * 2
            j0 = (r * 2) % 4
            fire(0, j0, b0)

            @pl.when(r < _ROUNDS - 1)
            def _():
                stage_idx(0, (j0 + 2) % 4, b0 + 2)

            @pl.when(r >= 1)
            def _():
                proc(1, b0 - 1, r >= 2)

            fire(1, j0 + 1, b0 + 1)

            @pl.when(r < _ROUNDS - 1)
            def _():
                stage_idx(1, (j0 + 3) % 4, b0 + 3)

            proc(0, b0, r >= 1)
            return carry

        stage_idx(0, 0, elem_base)
        stage_idx(1, 1, elem_base + 1)
        lax.fori_loop(0, _ROUNDS, round_body, 0)
        last = elem_base + _BPW - 1
        proc(1, last, True)
        drain_out(0, last - 1)
        drain_out(1, last)

    return _tec_body


def kernel(x, token_table, pos_table):
    tok128 = jnp.pad(token_table, ((0, 0), (0, _PADDED - _EMBED)))
    mesh = plsc.VectorSubcoreMesh(core_axis_name="c", subcore_axis_name="s")

    def part(t0, tlen):
        x1d = x[:, t0:t0 + tlen].astype(jnp.int32).reshape(-1)
        pos1d = pos_table[t0:t0 + tlen].reshape(-1)
        return pl.kernel(
            _make_tec_body(tlen),
            out_type=jax.ShapeDtypeStruct((_BATCH, tlen, _EMBED), jnp.float32),
            mesh=mesh,
            compiler_params=pltpu.CompilerParams(use_tc_tiling_on_sc=True),
            scratch_types=[
                pltpu.VMEM((tlen * _EMBED,), jnp.float32),
                pltpu.VMEM((4 * _ISTRIDE,), jnp.int32),
                pltpu.VMEM((2, tlen, _PADDED), jnp.float32),
                pltpu.VMEM((2, tlen, _EMBED), jnp.float32),
                pltpu.SemaphoreType.DMA,
                pltpu.SemaphoreType.DMA,
                pltpu.SemaphoreType.DMA,
                pltpu.SemaphoreType.DMA,
                pltpu.SemaphoreType.DMA,
                pltpu.SemaphoreType.DMA,
            ],
        )(x1d, tok128, pos1d)

    p1 = part(0, 96)
    p2 = part(96, 104)
    return jnp.concatenate([p1, p2], axis=1)


# speedup vs baseline: 1.2730x; 1.2730x over previous
import jax
import jax.numpy as jnp
from jax import lax
from jax.experimental import pallas as pl
from jax.experimental.pallas import tpu as pltpu
from jax.experimental.pallas import tpu_sc as plsc

_MAXLEN = 200
_EMBED = 64
_VOCAB = 100000
_BATCH = 4096
_LANES = 16
_PADDED = 128

_NC = 2
_NS = 16
_NW = _NC * _NS
_BPW = _BATCH // _NW
_ROUNDS = _BPW // 2
_J = _EMBED // _LANES


def _tec_body(x_hbm, tok_hbm, pos_hbm, out_hbm, pos_v, idx_v, rows_v, outs_v,
              g0, g1, o0, o1, i0, i1):
    c = lax.axis_index("c")
    s = lax.axis_index("s")
    wid = s * _NC + c
    gsems = (g0, g1)
    osems = (o0, o1)
    isems = (i0, i1)
    pltpu.sync_copy(pos_hbm, pos_v)
    elem_base = wid * _BPW

    def stage_idx(sem, j, b):
        pltpu.async_copy(x_hbm.at[pl.ds(b * _MAXLEN, _MAXLEN)],
                         idx_v.at[pl.ds(pl.multiple_of(j * _MAXLEN, 8),
                                        _MAXLEN)],
                         isems[sem])

    def fire(slot, j, b):
        i0 = pl.multiple_of(j * _MAXLEN, 8)
        pltpu.make_async_copy(x_hbm.at[pl.ds(b * _MAXLEN, _MAXLEN)],
                              idx_v.at[pl.ds(i0, _MAXLEN)],
                              isems[slot]).wait()
        pltpu.async_copy(tok_hbm.at[idx_v.at[pl.ds(i0, 128)]],
                         rows_v.at[slot, pl.ds(0, 128)], gsems[slot])
        pltpu.async_copy(tok_hbm.at[idx_v.at[pl.ds(i0 + 128, 72)]],
                         rows_v.at[slot, pl.ds(128, 72)], gsems[slot])

    def wait_gathers(slot):
        pltpu.make_async_copy(tok_hbm.at[pl.ds(0, _MAXLEN)],
                              rows_v.at[slot], gsems[slot]).wait()

    def drain_out(slot, b):
        pltpu.make_async_copy(outs_v.at[slot], out_hbm.at[b],
                              osems[slot]).wait()

    def add_positions(slot):
        @plsc.parallel_loop(0, _MAXLEN, unroll=4)
        def _(t):
            for jj in range(_J):
                p = pos_v[pl.ds(t * _EMBED + jj * _LANES, _LANES)]
                outs_v[slot, t, pl.ds(jj * _LANES, _LANES)] = (
                    rows_v[slot, t, pl.ds(jj * _LANES, _LANES)] + p
                )

    def proc(slot, b, drain_pred):
        wait_gathers(slot)

        @pl.when(drain_pred)
        def _():
            drain_out(slot, b - 2)

        add_positions(slot)
        pltpu.async_copy(outs_v.at[slot], out_hbm.at[b], osems[slot])

    def round_body(r, carry):
        b0 = elem_base + r * 2
        j0 = (r * 2) % 4
        fire(0, j0, b0)

        @pl.when(r < _ROUNDS - 1)
        def _():
            stage_idx(0, (j0 + 2) % 4, b0 + 2)

        @pl.when(r >= 1)
        def _():
            proc(1, b0 - 1, r >= 2)

        fire(1, j0 + 1, b0 + 1)

        @pl.when(r < _ROUNDS - 1)
        def _():
            stage_idx(1, (j0 + 3) % 4, b0 + 3)

        proc(0, b0, r >= 1)
        return carry

    stage_idx(0, 0, elem_base)
    stage_idx(1, 1, elem_base + 1)
    lax.fori_loop(0, _ROUNDS, round_body, 0)
    last = elem_base + _BPW - 1
    proc(1, last, True)
    drain_out(0, last - 1)
    drain_out(1, last)


def kernel(x, token_table, pos_table):
    x1d = x.reshape(-1).astype(jnp.int32)
    tok128 = jnp.pad(token_table, ((0, 0), (0, _PADDED - _EMBED)))
    pos1d = pos_table.reshape(-1)
    mesh = plsc.VectorSubcoreMesh(core_axis_name="c", subcore_axis_name="s")
    out = pl.kernel(
        _tec_body,
        out_type=jax.ShapeDtypeStruct((_BATCH, _MAXLEN, _EMBED), jnp.float32),
        mesh=mesh,
        compiler_params=pltpu.CompilerParams(use_tc_tiling_on_sc=True),
        scratch_types=[
            pltpu.VMEM((_MAXLEN * _EMBED,), jnp.float32),
            pltpu.VMEM((4 * _MAXLEN,), jnp.int32),
            pltpu.VMEM((2, _MAXLEN, _PADDED), jnp.float32),
            pltpu.VMEM((2, _MAXLEN, _EMBED), jnp.float32),
            pltpu.SemaphoreType.DMA,
            pltpu.SemaphoreType.DMA,
            pltpu.SemaphoreType.DMA,
            pltpu.SemaphoreType.DMA,
            pltpu.SemaphoreType.DMA,
            pltpu.SemaphoreType.DMA,
        ],
    )(x1d, tok128, pos1d)
    return out
